# baseline (speedup 1.0000x reference)
import functools

import jax
import jax.numpy as jnp
from jax import lax
from jax.experimental import pallas as pl
from jax.experimental.pallas import tpu as pltpu
from jax.experimental.pallas import tpu_sc as plsc

N = 10000
E = 320000
IN = 128
H = 32
C = 4
B = 64
OUT = 10
ALPHA = 0.5
DKV = 32

NC = 2
NS = 16
NW = NC * NS
K = 80
EW = E // NW
NCH = EW // K
ET = E // NS
NCH2 = ET // K
HN = N // NC
HNP = HN + 8
ROWS_T = 312
ROWS_L = HN - (NS - 1) * ROWS_T

BN = 1000
NG = N // BN

_f32 = jnp.float32


def _mesh():
    return plsc.VectorSubcoreMesh(
        core_axis_name="c", subcore_axis_name="s",
        num_cores=NC, num_subcores=NS)


@functools.lru_cache(maxsize=None)
def _build_deg(interpret=False):
    @functools.partial(
        pl.kernel,
        out_type=jax.ShapeDtypeStruct((NC, N), _f32),
        mesh=_mesh(),
        scratch_types=[
            pltpu.VMEM((NCH, K), jnp.int32),
            pltpu.VMEM((K,), _f32),
            pltpu.VMEM((2000,), _f32),
            pltpu.VMEM_SHARED((N,), _f32),
        ],
        interpret=interpret,
    )
    def deg_kernel(dstr, out, dst_v, ones_v, zbuf, acc):
        cid = lax.axis_index("c")
        sid = lax.axis_index("s")
        wid = cid * NS + sid

        @pl.loop(0, 2000 // 16)
        def _z(i):
            zbuf[pl.ds(i * 16, 16)] = jnp.zeros((16,), _f32)

        for i in range(K // 16):
            ones_v[pl.ds(i * 16, 16)] = jnp.ones((16,), _f32)

        @pl.when(sid == 0)
        def _():
            for q in range(N // 2000):
                pltpu.sync_copy(zbuf, acc.at[pl.ds(q * 2000, 2000)])

        pltpu.sync_copy(dstr.at[wid], dst_v)
        plsc.subcore_barrier()

        @pl.loop(0, NCH)
        def _s(j):
            pltpu.sync_copy(ones_v, acc.at[dst_v.at[j]], add=True)

        plsc.subcore_barrier()

        @pl.when(sid == 0)
        def _():
            pltpu.sync_copy(acc, out.at[cid])

    return deg_kernel


@functools.lru_cache(maxsize=None)
def _build_prop(W, interpret=False):
    @functools.partial(
        pl.kernel,
        out_type=jax.ShapeDtypeStruct((N, W), _f32),
        mesh=_mesh(),
        scratch_types=[
            pltpu.VMEM((NCH2, K), jnp.int32),
            pltpu.VMEM((NCH2, K), jnp.int32),
            pltpu.VMEM((K, W), _f32),
            pltpu.VMEM((K, W), _f32),
            pltpu.VMEM((8, W), _f32),
            pltpu.VMEM_SHARED((HNP, W), _f32),
            pltpu.SemaphoreType.DMA,
            pltpu.SemaphoreType.DMA,
        ],
        interpret=interpret,
    )
    def prop_kernel(table, srcr, dstr, out, src_v, dst_v, rows0, rows1,
                    zbuf, acc, sem0, sem1):
        cid = lax.axis_index("c")
        sid = lax.axis_index("s")

        for r in range(8):
            for q in range(W // 16):
                zbuf[r, pl.ds(q * 16, 16)] = jnp.zeros((16,), _f32)

        base = sid * ROWS_T
        nt8 = jnp.where(sid == NS - 1, ROWS_L // 8, ROWS_T // 8)

        @pl.loop(0, nt8)
        def _z(j):
            pltpu.sync_copy(zbuf, acc.at[pl.ds(base + j * 8, 8)])

        pltpu.sync_copy(srcr.at[sid], src_v)
        pltpu.sync_copy(dstr.at[sid], dst_v)

        lo = cid * HN

        @pl.loop(0, NCH2)
        def _m(r):
            for q in range(K // 16):
                d = dst_v[r, pl.ds(q * 16, 16)] - lo
                oob = (d < 0) | (d >= HN)
                dst_v[r, pl.ds(q * 16, 16)] = jnp.where(oob, HN, d)
                s = src_v[r, pl.ds(q * 16, 16)]
                src_v[r, pl.ds(q * 16, 16)] = jnp.where(oob, 0, s)

        plsc.subcore_barrier()

        def start(j, buf, sem):
            pltpu.async_copy(table.at[src_v.at[j]], buf, sem)

        def wait(buf, sem):
            pltpu.make_async_copy(table.at[src_v.at[0]], buf, sem).wait()

        def scat(j, buf):
            pltpu.sync_copy(buf, acc.at[dst_v.at[j]], add=True)

        start(0, rows0, sem0)

        @pl.loop(0, (NCH2 - 2) // 2)
        def _b(i):
            start(2 * i + 1, rows1, sem1)
            wait(rows0, sem0)
            scat(2 * i, rows0)
            start(2 * i + 2, rows0, sem0)
            wait(rows1, sem1)
            scat(2 * i + 1, rows1)

        start(NCH2 - 1, rows1, sem1)
        wait(rows0, sem0)
        scat(NCH2 - 2, rows0)
        wait(rows1, sem1)
        scat(NCH2 - 1, rows1)

        plsc.subcore_barrier()

        @pl.when(sid < NS - 1)
        def _():
            pltpu.sync_copy(acc.at[pl.ds(base, ROWS_T)],
                            out.at[pl.ds(lo + base, ROWS_T)])

        @pl.when(sid == NS - 1)
        def _():
            pltpu.sync_copy(acc.at[pl.ds((NS - 1) * ROWS_T, ROWS_L)],
                            out.at[pl.ds(lo + (NS - 1) * ROWS_T, ROWS_L)])

    return prop_kernel


def _embed_body(data, w1, b1, w2, b2, wg0, d0, d1, x_o, g_o, rs_o):
    rs = lax.rsqrt(d0[...] + d1[...] + 1.0)
    x = jnp.dot(data[...], w1[...], preferred_element_type=_f32) + b1[...]
    x = jnp.dot(x, w2[...], preferred_element_type=_f32) + b2[...]
    x_o[...] = x
    rs_o[...] = rs
    for c in range(C):
        g_o[:, c * H:(c + 1) * H] = (
            jnp.dot(x, wg0[c], preferred_element_type=_f32) * rs)


@functools.lru_cache(maxsize=None)
def _build_embed(interpret=False):
    row = lambda i: (i, 0)
    full = lambda i: (0, 0)
    full3 = lambda i: (0, 0, 0)
    return pl.pallas_call(
        _embed_body,
        grid=(NG,),
        in_specs=[
            pl.BlockSpec((BN, IN), row),
            pl.BlockSpec((IN, H), full),
            pl.BlockSpec((1, H), full),
            pl.BlockSpec((H, H), full),
            pl.BlockSpec((1, H), full),
            pl.BlockSpec((C, H, H), full3),
            pl.BlockSpec((BN, 1), row),
            pl.BlockSpec((BN, 1), row),
        ],
        out_specs=[
            pl.BlockSpec((BN, H), row),
            pl.BlockSpec((BN, C * H), row),
            pl.BlockSpec((BN, 1), row),
        ],
        out_shape=[
            jax.ShapeDtypeStruct((N, H), _f32),
            jax.ShapeDtypeStruct((N, C * H), _f32),
            jax.ShapeDtypeStruct((N, 1), _f32),
        ],
        interpret=interpret,
    )


def _layer0_body(p, x, rs, bg0, wg1, h_o, g_o):
    agg = p[...] * rs[...]
    x_b = x[...]
    for c in range(C):
        sl = slice(c * H, (c + 1) * H)
        hn = jnp.maximum(agg[:, sl] + bg0[c], 0.0)
        h = ALPHA * x_b + (1.0 - ALPHA) * hn
        g = jnp.dot(h, wg1[c], preferred_element_type=_f32) * rs[...]
        h_o[:, sl] = h
        g_o[:, sl] = g


@functools.lru_cache(maxsize=None)
def _build_layer0(interpret=False):
    row = lambda i: (i, 0)
    full = lambda i: (0, 0)
    full3 = lambda i: (0, 0, 0)
    return pl.pallas_call(
        _layer0_body,
        grid=(NG,),
        in_specs=[
            pl.BlockSpec((BN, C * H), row),
            pl.BlockSpec((BN, H), row),
            pl.BlockSpec((BN, 1), row),
            pl.BlockSpec((C, H), full),
            pl.BlockSpec((C, H, H), full3),
        ],
        out_specs=[
            pl.BlockSpec((BN, C * H), row),
            pl.BlockSpec((BN, C * H), row),
        ],
        out_shape=[
            jax.ShapeDtypeStruct((N, C * H), _f32),
            jax.ShapeDtypeStruct((N, C * H), _f32),
        ],
        interpret=interpret,
    )


def _layer1_body(p, h, rs, wgn, bgl, h_o, g_o):
    agg = p[...] * rs[...]
    for c in range(C):
        sl = slice(c * H, (c + 1) * H)
        hn = jnp.maximum(agg[:, sl] + bgl[c], 0.0)
        hc = ALPHA * h[:, sl] + (1.0 - ALPHA) * hn
        g = jnp.dot(hc, wgn[c], preferred_element_type=_f32) * rs[...]
        h_o[:, sl] = hc
        g_o[:, sl] = g


@functools.lru_cache(maxsize=None)
def _build_layer1(interpret=False):
    row = lambda i: (i, 0)
    full = lambda i: (0, 0)
    full3 = lambda i: (0, 0, 0)
    return pl.pallas_call(
        _layer1_body,
        grid=(NG,),
        in_specs=[
            pl.BlockSpec((BN, C * H), row),
            pl.BlockSpec((BN, C * H), row),
            pl.BlockSpec((BN, 1), row),
            pl.BlockSpec((C, H, H), full3),
            pl.BlockSpec((C, H), full),
        ],
        out_specs=[
            pl.BlockSpec((BN, C * H), row),
            pl.BlockSpec((BN, C * H), row),
        ],
        out_shape=[
            jax.ShapeDtypeStruct((N, C * H), _f32),
            jax.ShapeDtypeStruct((N, C * H), _f32),
        ],
        interpret=interpret,
    )


def _layer2_body(p, h, rs, bgl, wa, var, h_o, s_o):
    agg = p[...] * rs[...]
    scs = []
    for c in range(C):
        sl = slice(c * H, (c + 1) * H)
        hn = jnp.maximum(agg[:, sl] + bgl[c], 0.0)
        hc = ALPHA * h[:, sl] + (1.0 - ALPHA) * hn
        t = jnp.tanh(jnp.dot(hc, wa[c], preferred_element_type=_f32))
        scs.append(jnp.dot(t, var[c], preferred_element_type=_f32))
        h_o[:, sl] = hc
    s_o[...] = jnp.concatenate(scs, axis=1)


@functools.lru_cache(maxsize=None)
def _build_layer2(interpret=False):
    row = lambda i: (i, 0)
    full = lambda i: (0, 0)
    full3 = lambda i: (0, 0, 0)
    return pl.pallas_call(
        _layer2_body,
        grid=(NG,),
        in_specs=[
            pl.BlockSpec((BN, C * H), row),
            pl.BlockSpec((BN, C * H), row),
            pl.BlockSpec((BN, 1), row),
            pl.BlockSpec((C, H), full),
            pl.BlockSpec((C, H, H), full3),
            pl.BlockSpec((C, H, 1), full3),
        ],
        out_specs=[
            pl.BlockSpec((BN, C * H), row),
            pl.BlockSpec((BN, C), row),
        ],
        out_shape=[
            jax.ShapeDtypeStruct((N, C * H), _f32),
            jax.ShapeDtypeStruct((N, C), _f32),
        ],
        interpret=interpret,
    )


def _pool_body(h, s, bat, z_o):
    def seg_onehot(i):
        bb = bat[pl.ds(i * BN, BN), :]
        cols = lax.broadcasted_iota(jnp.int32, (BN, B), 1)
        return (bb == cols).astype(_f32)

    def max_step(i, m):
        oh = seg_onehot(i)
        sb = s[pl.ds(i * BN, BN), :]
        upd = []
        for c in range(C):
            v = jnp.where(oh > 0.0, sb[:, c:c + 1], -1e30)
            upd.append(jnp.max(v, axis=0, keepdims=True).T)
        return jnp.maximum(m, jnp.concatenate(upd, axis=1))

    m = lax.fori_loop(0, NG, max_step,
                      jnp.full((B, C), -1e30, _f32))

    def sum_step(i, carry):
        ssum, pool = carry
        oh = seg_onehot(i)
        sb = s[pl.ds(i * BN, BN), :]
        hb = h[pl.ds(i * BN, BN), :]
        mb = jnp.dot(oh, m, preferred_element_type=_f32)
        e = jnp.exp(sb - mb)
        dn = (((0,), (0,)), ((), ()))
        ssum = ssum + lax.dot_general(oh, e, dn,
                                      preferred_element_type=_f32)
        pc = []
        for c in range(C):
            w = oh * e[:, c:c + 1]
            pc.append(lax.dot_general(w, hb[:, c * H:(c + 1) * H], dn,
                                      preferred_element_type=_f32))
        return ssum, pool + jnp.concatenate(pc, axis=1)

    ssum, pool = lax.fori_loop(
        0, NG, sum_step,
        (jnp.zeros((B, C), _f32), jnp.zeros((B, C * H), _f32)))

    zs = []
    for c in range(C):
        zs.append(pool[:, c * H:(c + 1) * H] / (ssum[:, c:c + 1] + 1e-9))
    z_o[...] = jnp.concatenate(zs, axis=1)


@functools.lru_cache(maxsize=None)
def _build_pool(interpret=False):
    return pl.pallas_call(
        _pool_body,
        out_shape=jax.ShapeDtypeStruct((B, C * H), _f32),
        interpret=interpret,
    )


def _head_body(z, wq, wk, wc1, bc1, wc2, bc2, o):
    zb = z[...]
    qs = [jnp.dot(zb[:, c * H:(c + 1) * H], wq[...],
                  preferred_element_type=_f32) for c in range(C)]
    ks = [jnp.dot(zb[:, c * H:(c + 1) * H], wk[...],
                  preferred_element_type=_f32) for c in range(C)]
    inv = 1.0 / jnp.sqrt(jnp.float32(DKV))
    z2 = []
    for c in range(C):
        lg = [jnp.sum(qs[c] * ks[e], axis=1, keepdims=True) * inv
              for e in range(C)]
        lg = jnp.concatenate(lg, axis=1)
        mx = jnp.max(lg, axis=1, keepdims=True)
        ex = jnp.exp(lg - mx)
        at = ex / jnp.sum(ex, axis=1, keepdims=True)
        acc = jnp.zeros((B, H), _f32)
        for e in range(C):
            acc = acc + at[:, e:e + 1] * zb[:, e * H:(e + 1) * H]
        z2.append(acc)
    flat = jnp.concatenate(z2, axis=1)
    hh = jnp.maximum(
        jnp.dot(flat, wc1[...], preferred_element_type=_f32) + bc1[...], 0.0)
    o[...] = jnp.dot(hh, wc2[...], preferred_element_type=_f32) + bc2[...]


@functools.lru_cache(maxsize=None)
def _build_head(interpret=False):
    return pl.pallas_call(
        _head_body,
        out_shape=jax.ShapeDtypeStruct((B, OUT), _f32),
        interpret=interpret,
    )


def kernel(data, edge_index, batch, W1, b1, W2, b2, Wg, bg, Wa, va,
           Wq, Wk, Wc1, bc1, Wc2, bc2):
    src = edge_index[0].astype(jnp.int32)
    dst = edge_index[1].astype(jnp.int32)
    dst32 = dst.reshape(NW, NCH, K)
    src16 = src.reshape(NS, NCH2, K)
    dst16 = dst.reshape(NS, NCH2, K)

    degp = _build_deg()(dst32)
    d0 = degp[0].reshape(N, 1)
    d1 = degp[1].reshape(N, 1)

    x, g, rs = _build_embed()(
        data, W1, b1.reshape(1, H), W2, b2.reshape(1, H), Wg[:, 0], d0, d1)

    p = _build_prop(C * H)(g, src16, dst16)
    h, g = _build_layer0()(p, x, rs, bg[:, 0], Wg[:, 1])

    p = _build_prop(C * H)(g, src16, dst16)
    h, g = _build_layer1()(p, h, rs, Wg[:, 2], bg[:, 1])

    p = _build_prop(C * H)(g, src16, dst16)
    h, s = _build_layer2()(p, h, rs, bg[:, 2], Wa,
                           va.reshape(C, H, 1))

    z = _build_pool()(h, s, batch.astype(jnp.int32).reshape(N, 1))
    out = _build_head()(z, Wq, Wk, Wc1, bc1.reshape(1, 64),
                        Wc2, bc2.reshape(1, OUT))
    return out


# speedup vs baseline: 45.3604x; 45.3604x over previous
import functools

import jax
import jax.numpy as jnp
from jax import lax
from jax.experimental import pallas as pl
from jax.experimental.pallas import tpu as pltpu
from jax.experimental.pallas import tpu_sc as plsc

N = 10000
E = 320000
IN = 128
H = 32
C = 4
B = 64
OUT = 10
ALPHA = 0.5
DKV = 32

NC = 2
NS = 16
NW = NC * NS
K = 80
EW = E // NW
NCH = EW // K
ET = E // NS
NCH2 = ET // K
HN = N // NC
HNP = HN + 8
ROWS_T = 312
ROWS_L = HN - (NS - 1) * ROWS_T

CAP = 16 + ET + 144

BN = 1000
NG = N // BN

_f32 = jnp.float32


def _mesh():
    return plsc.VectorSubcoreMesh(
        core_axis_name="c", subcore_axis_name="s",
        num_cores=NC, num_subcores=NS)


@functools.lru_cache(maxsize=None)
def _build_deg(interpret=False):
    @functools.partial(
        pl.kernel,
        out_type=jax.ShapeDtypeStruct((NC, N), _f32),
        mesh=_mesh(),
        scratch_types=[
            pltpu.VMEM((NCH, K), jnp.int32),
            pltpu.VMEM((K,), _f32),
            pltpu.VMEM((2000,), _f32),
            pltpu.VMEM_SHARED((N,), _f32),
        ],
        interpret=interpret,
    )
    def deg_kernel(dstr, out, dst_v, ones_v, zbuf, acc):
        cid = lax.axis_index("c")
        sid = lax.axis_index("s")
        wid = cid * NS + sid

        @pl.loop(0, 2000 // 16)
        def _z(i):
            zbuf[pl.ds(i * 16, 16)] = jnp.zeros((16,), _f32)

        for i in range(K // 16):
            ones_v[pl.ds(i * 16, 16)] = jnp.ones((16,), _f32)

        @pl.when(sid == 0)
        def _():
            for q in range(N // 2000):
                pltpu.sync_copy(zbuf, acc.at[pl.ds(q * 2000, 2000)])

        pltpu.sync_copy(dstr.at[wid], dst_v)
        plsc.subcore_barrier()

        @pl.loop(0, NCH)
        def _s(j):
            pltpu.sync_copy(ones_v, acc.at[dst_v.at[j]], add=True)

        plsc.subcore_barrier()

        @pl.when(sid == 0)
        def _():
            pltpu.sync_copy(acc, out.at[cid])

    return deg_kernel


@functools.lru_cache(maxsize=None)
def _build_part(interpret=False):
    @functools.partial(
        pl.kernel,
        out_type=[
            jax.ShapeDtypeStruct((NC, NS, CAP), jnp.int32),
            jax.ShapeDtypeStruct((NC, NS, CAP), jnp.int32),
        ],
        mesh=_mesh(),
        scratch_types=[
            pltpu.VMEM((NCH2, K), jnp.int32),
            pltpu.VMEM((NCH2, K), jnp.int32),
            pltpu.VMEM((CAP,), jnp.int32),
            pltpu.VMEM((CAP,), jnp.int32),
        ],
        compiler_params=pltpu.CompilerParams(needs_layout_passes=False),
        interpret=interpret,
    )
    def part_kernel(srcr, dstr, srcp, dstp, src_v, dst_v, sc_c, ds_c):
        cid = lax.axis_index("c")
        sid = lax.axis_index("s")
        lo = cid * HN

        pltpu.sync_copy(srcr.at[sid], src_v)
        pltpu.sync_copy(dstr.at[sid], dst_v)

        @pl.loop(0, CAP // 16)
        def _t(i):
            sc_c[pl.ds(i * 16, 16)] = jnp.zeros((16,), jnp.int32)
            ds_c[pl.ds(i * 16, 16)] = jnp.full((16,), HN, jnp.int32)

        trash = jnp.int32(CAP - 8)

        @pl.loop(0, NCH2, init_carry=jnp.int32(16))
        def off(r, o):
            for q in range(K // 16):
                s = src_v[r, pl.ds(q * 16, 16)]
                d = dst_v[r, pl.ds(q * 16, 16)] - lo
                own = (d >= 0) & (d < HN)
                cum = plsc.cumsum(own.astype(jnp.int32))
                pos = jnp.where(own, o + cum - 1, trash)
                plsc.store_scatter(sc_c, [pos], s)
                plsc.store_scatter(ds_c, [pos], jnp.where(own, d, HN))
                o = o + jnp.max(cum)
            return o

        cnt = off - 16
        sc_c[pl.ds(0, 16)] = jnp.broadcast_to(cnt, (16,)).astype(jnp.int32)

        pltpu.sync_copy(sc_c, srcp.at[cid, sid])
        pltpu.sync_copy(ds_c, dstp.at[cid, sid])

    return part_kernel


@functools.lru_cache(maxsize=None)
def _build_prop(W, interpret=False):
    @functools.partial(
        pl.kernel,
        out_type=jax.ShapeDtypeStruct((N, W), _f32),
        mesh=_mesh(),
        scratch_types=[
            pltpu.VMEM((CAP,), jnp.int32),
            pltpu.VMEM((CAP,), jnp.int32),
            pltpu.VMEM((K, W), _f32),
            pltpu.VMEM((K, W), _f32),
            pltpu.VMEM((104, W), _f32),
            pltpu.VMEM((K,), jnp.int32),
            pltpu.VMEM_SHARED((HNP, W), _f32),
            pltpu.SemaphoreType.DMA,
            pltpu.SemaphoreType.DMA,
        ],
        compiler_params=pltpu.CompilerParams(needs_layout_passes=False),
        interpret=interpret,
    )
    def prop_kernel(table, srcp, dstp, out, srcc, dstc, rows0, rows1,
                    zbuf, dstm, acc, sem0, sem1):
        cid = lax.axis_index("c")
        sid = lax.axis_index("s")
        lo = cid * HN

        pltpu.sync_copy(srcp.at[cid, sid], srcc)
        pltpu.sync_copy(dstp.at[cid, sid], dstc)

        @pl.loop(0, 104)
        def _zf(r):
            for q in range(W // 16):
                zbuf[r, pl.ds(q * 16, 16)] = jnp.zeros((16,), _f32)

        base = sid * ROWS_T
        for j in range(3):
            pltpu.sync_copy(zbuf.at[pl.ds(0, 104)],
                            acc.at[pl.ds(base + j * 104, 104)])

        @pl.when(sid == NS - 1)
        def _():
            pltpu.sync_copy(zbuf.at[pl.ds(0, ROWS_L - 3 * 104)],
                            acc.at[pl.ds(base + 312, ROWS_L - 3 * 104)])

        c16 = srcc[pl.ds(0, 16)]
        cnt = jnp.sum(jnp.where(lax.iota(jnp.int32, 16) == 0, c16, 0))
        nch = (cnt + (K - 1)) // K
        mc = jnp.maximum(nch + nch % 2, 2)

        plsc.subcore_barrier()

        def start(j, buf, sem):
            pltpu.async_copy(table.at[srcc.at[pl.ds(16 + j * K, K)]],
                             buf, sem)

        def wait(buf, sem):
            pltpu.make_async_copy(table.at[srcc.at[pl.ds(16, K)]],
                                  buf, sem).wait()

        def scat(j, buf):
            for q in range(K // 16):
                dstm[pl.ds(q * 16, 16)] = dstc[pl.ds(16 + j * K + q * 16, 16)]
            pltpu.sync_copy(buf, acc.at[dstm], add=True)

        start(0, rows0, sem0)

        @pl.loop(0, mc // 2 - 1)
        def _b(i):
            start(2 * i + 1, rows1, sem1)
            wait(rows0, sem0)
            scat(2 * i, rows0)
            start(2 * i + 2, rows0, sem0)
            wait(rows1, sem1)
            scat(2 * i + 1, rows1)

        start(mc - 1, rows1, sem1)
        wait(rows0, sem0)
        scat(mc - 2, rows0)
        wait(rows1, sem1)
        scat(mc - 1, rows1)

        plsc.subcore_barrier()

        @pl.when(sid < NS - 1)
        def _():
            pltpu.sync_copy(acc.at[pl.ds(base, ROWS_T)],
                            out.at[pl.ds(lo + base, ROWS_T)])

        @pl.when(sid == NS - 1)
        def _():
            pltpu.sync_copy(acc.at[pl.ds((NS - 1) * ROWS_T, ROWS_L)],
                            out.at[pl.ds(lo + (NS - 1) * ROWS_T, ROWS_L)])

    return prop_kernel


def _embed_body(data, w1, b1, w2, b2, wg0, d0, d1, x_o, g_o, rs_o):
    rs = lax.rsqrt(d0[...] + d1[...] + 1.0)
    x = jnp.dot(data[...], w1[...], preferred_element_type=_f32) + b1[...]
    x = jnp.dot(x, w2[...], preferred_element_type=_f32) + b2[...]
    x_o[...] = x
    rs_o[...] = rs
    for c in range(C):
        g_o[:, c * H:(c + 1) * H] = (
            jnp.dot(x, wg0[c], preferred_element_type=_f32) * rs)


@functools.lru_cache(maxsize=None)
def _build_embed(interpret=False):
    row = lambda i: (i, 0)
    full = lambda i: (0, 0)
    full3 = lambda i: (0, 0, 0)
    return pl.pallas_call(
        _embed_body,
        grid=(NG,),
        in_specs=[
            pl.BlockSpec((BN, IN), row),
            pl.BlockSpec((IN, H), full),
            pl.BlockSpec((1, H), full),
            pl.BlockSpec((H, H), full),
            pl.BlockSpec((1, H), full),
            pl.BlockSpec((C, H, H), full3),
            pl.BlockSpec((BN, 1), row),
            pl.BlockSpec((BN, 1), row),
        ],
        out_specs=[
            pl.BlockSpec((BN, H), row),
            pl.BlockSpec((BN, C * H), row),
            pl.BlockSpec((BN, 1), row),
        ],
        out_shape=[
            jax.ShapeDtypeStruct((N, H), _f32),
            jax.ShapeDtypeStruct((N, C * H), _f32),
            jax.ShapeDtypeStruct((N, 1), _f32),
        ],
        interpret=interpret,
    )


def _layer0_body(p, x, rs, bg0, wg1, h_o, g_o):
    agg = p[...] * rs[...]
    x_b = x[...]
    for c in range(C):
        sl = slice(c * H, (c + 1) * H)
        hn = jnp.maximum(agg[:, sl] + bg0[c], 0.0)
        h = ALPHA * x_b + (1.0 - ALPHA) * hn
        g = jnp.dot(h, wg1[c], preferred_element_type=_f32) * rs[...]
        h_o[:, sl] = h
        g_o[:, sl] = g


@functools.lru_cache(maxsize=None)
def _build_layer0(interpret=False):
    row = lambda i: (i, 0)
    full = lambda i: (0, 0)
    full3 = lambda i: (0, 0, 0)
    return pl.pallas_call(
        _layer0_body,
        grid=(NG,),
        in_specs=[
            pl.BlockSpec((BN, C * H), row),
            pl.BlockSpec((BN, H), row),
            pl.BlockSpec((BN, 1), row),
            pl.BlockSpec((C, H), full),
            pl.BlockSpec((C, H, H), full3),
        ],
        out_specs=[
            pl.BlockSpec((BN, C * H), row),
            pl.BlockSpec((BN, C * H), row),
        ],
        out_shape=[
            jax.ShapeDtypeStruct((N, C * H), _f32),
            jax.ShapeDtypeStruct((N, C * H), _f32),
        ],
        interpret=interpret,
    )


def _layer1_body(p, h, rs, wgn, bgl, h_o, g_o):
    agg = p[...] * rs[...]
    for c in range(C):
        sl = slice(c * H, (c + 1) * H)
        hn = jnp.maximum(agg[:, sl] + bgl[c], 0.0)
        hc = ALPHA * h[:, sl] + (1.0 - ALPHA) * hn
        g = jnp.dot(hc, wgn[c], preferred_element_type=_f32) * rs[...]
        h_o[:, sl] = hc
        g_o[:, sl] = g


@functools.lru_cache(maxsize=None)
def _build_layer1(interpret=False):
    row = lambda i: (i, 0)
    full = lambda i: (0, 0)
    full3 = lambda i: (0, 0, 0)
    return pl.pallas_call(
        _layer1_body,
        grid=(NG,),
        in_specs=[
            pl.BlockSpec((BN, C * H), row),
            pl.BlockSpec((BN, C * H), row),
            pl.BlockSpec((BN, 1), row),
            pl.BlockSpec((C, H, H), full3),
            pl.BlockSpec((C, H), full),
        ],
        out_specs=[
            pl.BlockSpec((BN, C * H), row),
            pl.BlockSpec((BN, C * H), row),
        ],
        out_shape=[
            jax.ShapeDtypeStruct((N, C * H), _f32),
            jax.ShapeDtypeStruct((N, C * H), _f32),
        ],
        interpret=interpret,
    )


def _layer2_body(p, h, rs, bgl, wa, var, h_o, s_o):
    agg = p[...] * rs[...]
    scs = []
    for c in range(C):
        sl = slice(c * H, (c + 1) * H)
        hn = jnp.maximum(agg[:, sl] + bgl[c], 0.0)
        hc = ALPHA * h[:, sl] + (1.0 - ALPHA) * hn
        t = jnp.tanh(jnp.dot(hc, wa[c], preferred_element_type=_f32))
        scs.append(jnp.dot(t, var[c], preferred_element_type=_f32))
        h_o[:, sl] = hc
    s_o[...] = jnp.concatenate(scs, axis=1)


@functools.lru_cache(maxsize=None)
def _build_layer2(interpret=False):
    row = lambda i: (i, 0)
    full = lambda i: (0, 0)
    full3 = lambda i: (0, 0, 0)
    return pl.pallas_call(
        _layer2_body,
        grid=(NG,),
        in_specs=[
            pl.BlockSpec((BN, C * H), row),
            pl.BlockSpec((BN, C * H), row),
            pl.BlockSpec((BN, 1), row),
            pl.BlockSpec((C, H), full),
            pl.BlockSpec((C, H, H), full3),
            pl.BlockSpec((C, H, 1), full3),
        ],
        out_specs=[
            pl.BlockSpec((BN, C * H), row),
            pl.BlockSpec((BN, C), row),
        ],
        out_shape=[
            jax.ShapeDtypeStruct((N, C * H), _f32),
            jax.ShapeDtypeStruct((N, C), _f32),
        ],
        interpret=interpret,
    )


def _pool_body(h, s, bat, z_o):
    def seg_onehot(i):
        bb = bat[pl.ds(i * BN, BN), :]
        cols = lax.broadcasted_iota(jnp.int32, (BN, B), 1)
        return (bb == cols).astype(_f32)

    def max_step(i, m):
        oh = seg_onehot(i)
        sb = s[pl.ds(i * BN, BN), :]
        upd = []
        for c in range(C):
            v = jnp.where(oh > 0.0, sb[:, c:c + 1], -1e30)
            upd.append(jnp.max(v, axis=0, keepdims=True).T)
        return jnp.maximum(m, jnp.concatenate(upd, axis=1))

    m = lax.fori_loop(0, NG, max_step,
                      jnp.full((B, C), -1e30, _f32))

    def sum_step(i, carry):
        ssum, pool = carry
        oh = seg_onehot(i)
        sb = s[pl.ds(i * BN, BN), :]
        hb = h[pl.ds(i * BN, BN), :]
        mb = jnp.dot(oh, m, preferred_element_type=_f32)
        e = jnp.exp(sb - mb)
        dn = (((0,), (0,)), ((), ()))
        ssum = ssum + lax.dot_general(oh, e, dn,
                                      preferred_element_type=_f32)
        pc = []
        for c in range(C):
            w = oh * e[:, c:c + 1]
            pc.append(lax.dot_general(w, hb[:, c * H:(c + 1) * H], dn,
                                      preferred_element_type=_f32))
        return ssum, pool + jnp.concatenate(pc, axis=1)

    ssum, pool = lax.fori_loop(
        0, NG, sum_step,
        (jnp.zeros((B, C), _f32), jnp.zeros((B, C * H), _f32)))

    zs = []
    for c in range(C):
        zs.append(pool[:, c * H:(c + 1) * H] / (ssum[:, c:c + 1] + 1e-9))
    z_o[...] = jnp.concatenate(zs, axis=1)


@functools.lru_cache(maxsize=None)
def _build_pool(interpret=False):
    return pl.pallas_call(
        _pool_body,
        out_shape=jax.ShapeDtypeStruct((B, C * H), _f32),
        interpret=interpret,
    )


def _head_body(z, wq, wk, wc1, bc1, wc2, bc2, o):
    zb = z[...]
    qs = [jnp.dot(zb[:, c * H:(c + 1) * H], wq[...],
                  preferred_element_type=_f32) for c in range(C)]
    ks = [jnp.dot(zb[:, c * H:(c + 1) * H], wk[...],
                  preferred_element_type=_f32) for c in range(C)]
    inv = 1.0 / jnp.sqrt(jnp.float32(DKV))
    z2 = []
    for c in range(C):
        lg = [jnp.sum(qs[c] * ks[e], axis=1, keepdims=True) * inv
              for e in range(C)]
        lg = jnp.concatenate(lg, axis=1)
        mx = jnp.max(lg, axis=1, keepdims=True)
        ex = jnp.exp(lg - mx)
        at = ex / jnp.sum(ex, axis=1, keepdims=True)
        acc = jnp.zeros((B, H), _f32)
        for e in range(C):
            acc = acc + at[:, e:e + 1] * zb[:, e * H:(e + 1) * H]
        z2.append(acc)
    flat = jnp.concatenate(z2, axis=1)
    hh = jnp.maximum(
        jnp.dot(flat, wc1[...], preferred_element_type=_f32) + bc1[...], 0.0)
    o[...] = jnp.dot(hh, wc2[...], preferred_element_type=_f32) + bc2[...]


@functools.lru_cache(maxsize=None)
def _build_head(interpret=False):
    return pl.pallas_call(
        _head_body,
        out_shape=jax.ShapeDtypeStruct((B, OUT), _f32),
        interpret=interpret,
    )


def kernel(data, edge_index, batch, W1, b1, W2, b2, Wg, bg, Wa, va,
           Wq, Wk, Wc1, bc1, Wc2, bc2):
    src = edge_index[0].astype(jnp.int32)
    dst = edge_index[1].astype(jnp.int32)
    dst32 = dst.reshape(NW, NCH, K)
    src16 = src.reshape(NS, NCH2, K)
    dst16 = dst.reshape(NS, NCH2, K)

    degp = _build_deg()(dst32)
    d0 = degp[0].reshape(N, 1)
    d1 = degp[1].reshape(N, 1)

    srcp, dstp = _build_part()(src16, dst16)

    x, g, rs = _build_embed()(
        data, W1, b1.reshape(1, H), W2, b2.reshape(1, H), Wg[:, 0], d0, d1)

    p = _build_prop(C * H)(g, srcp, dstp)
    h, g = _build_layer0()(p, x, rs, bg[:, 0], Wg[:, 1])

    p = _build_prop(C * H)(g, srcp, dstp)
    h, g = _build_layer1()(p, h, rs, Wg[:, 2], bg[:, 1])

    p = _build_prop(C * H)(g, srcp, dstp)
    h, s = _build_layer2()(p, h, rs, bg[:, 2], Wa,
                           va.reshape(C, H, 1))

    z = _build_pool()(h, s, batch.astype(jnp.int32).reshape(N, 1))
    out = _build_head()(z, Wq, Wk, Wc1, bc1.reshape(1, 64),
                        Wc2, bc2.reshape(1, OUT))
    return out
